# baseline (speedup 1.0000x reference)
import jax
import jax.numpy as jnp
from jax import lax
from jax.experimental import pallas as pl
from jax.experimental.pallas import tpu as pltpu
from jax.experimental.pallas import tpu_sc as plsc

B = 4
NY = 512
NX = 512
C = 64
S = B * NY * NX
P = 40000
L = 16
NC = 2
NS = 16
NW = NC * NS
CW = 128
CHUNK = 128
NUM_CHUNKS = (P + NW * CHUNK - 1) // (NW * CHUNK) * NW
WAVE = 5

A_ROWS = 3 * S // 4
B_ROWS = S // 4
PAD = 8
KA = NUM_CHUNKS // NW
KB = NUM_CHUNKS // NS

FILL_ROWS = 128
TILE_B_ROWS = B_ROWS // NC // NS


def _lin_group(cbuf, k, g):
    bv = cbuf[k, 0, pl.ds(g * L, L)]
    yv = cbuf[k, 1, pl.ds(g * L, L)]
    xv = cbuf[k, 2, pl.ds(g * L, L)]
    return jnp.minimum(bv, B - 1) * (NY * NX) + yv * NX + xv


def _stage_and_scatter(vf_hbm, cols, out_hbm, cbuf, dbuf, ibuf, sem_in, sem_sc,
                       nchunks, chunk_of_k, route):
    for wave in range(nchunks // WAVE):
        in_copies = []
        for kk in range(WAVE):
            k = wave * WAVE + kk
            chunk = chunk_of_k(k)
            start = jnp.minimum(chunk * CHUNK, P - CHUNK)
            for j, col in enumerate(cols):
                in_copies.append(
                    pltpu.async_copy(col.at[pl.ds(start, CHUNK)],
                                     cbuf.at[k, j], sem_in))
            in_copies.append(
                pltpu.async_copy(vf_hbm.at[pl.ds(start, CHUNK)],
                                 dbuf.at[kk, :, pl.ds(0, C)], sem_in))
        for cp in in_copies:
            cp.wait()

        sc_copies = []
        for kk in range(WAVE):
            k = wave * WAVE + kk
            for g in range(CHUNK // L):
                ibuf[k, pl.ds(g * L, L)] = route(_lin_group(cbuf, k, g))
            sc_copies.append(
                pltpu.async_copy(dbuf.at[kk], out_hbm.at[ibuf.at[k]], sem_sc))
        for cp in sc_copies:
            cp.wait()


def _sc_scatter_a_body(vf_hbm, b_hbm, y_hbm, x_hbm, out_hbm,
                       cbuf, dbuf, ibuf, sem_in, sem_sc):
    cid = lax.axis_index("c")
    sid = lax.axis_index("s")
    w = sid * NC + cid

    def route(lin):
        return jnp.where(lin < A_ROWS, lin, A_ROWS)

    _stage_and_scatter(vf_hbm, (b_hbm, y_hbm, x_hbm), out_hbm,
                       cbuf, dbuf, ibuf, sem_in, sem_sc,
                       KA, lambda k: w + NW * k, route)


def _sc_fill_scatter_b_body(vf_hbm, b_hbm, y_hbm, x_hbm, out_hbm,
                            cbuf, dbuf, ibuf, zbuf, sem_in, sem_sc, sem_z):
    cid = lax.axis_index("c")
    sid = lax.axis_index("s")

    def zrow(j, _):
        for g in range(CW // L):
            zbuf[j, pl.ds(g * L, L)] = jnp.zeros((L,), jnp.float32)
        return _
    lax.fori_loop(0, FILL_ROWS, zrow, None)
    base = (cid * NS + sid) * TILE_B_ROWS
    for grp in range(TILE_B_ROWS // FILL_ROWS // 8):
        cps = [pltpu.async_copy(
                   zbuf,
                   out_hbm.at[pl.ds(base + (grp * 8 + i) * FILL_ROWS,
                                    FILL_ROWS)],
                   sem_z)
               for i in range(8)]
        for cp in cps:
            cp.wait()

    plsc.subcore_barrier()

    half = B_ROWS // NC
    lo = cid * half

    def route(lin):
        lin_b = lin - A_ROWS
        owned = (lin_b >= lo) & (lin_b < lo + half)
        return jnp.where(owned, lin_b, B_ROWS)

    _stage_and_scatter(vf_hbm, (b_hbm, y_hbm, x_hbm), out_hbm,
                       cbuf, dbuf, ibuf, sem_in, sem_sc,
                       KB, lambda k: sid + NS * k, route)


def _sc_scatter_a(vf, bcol, ycol, xcol):
    mesh = plsc.VectorSubcoreMesh(core_axis_name="c", subcore_axis_name="s")
    kfn = pl.kernel(
        _sc_scatter_a_body,
        mesh=mesh,
        out_type=(),
        compiler_params=pltpu.CompilerParams(use_tc_tiling_on_sc=False),
        scratch_types=[
            pltpu.VMEM((KA, 3, CHUNK), jnp.int32),
            pltpu.VMEM((WAVE, CHUNK, CW), jnp.float32),
            pltpu.VMEM((KA, CHUNK), jnp.int32),
            pltpu.SemaphoreType.DMA,
            pltpu.SemaphoreType.DMA,
        ],
    )
    canvas_ref = jax.new_ref(jnp.zeros((A_ROWS + PAD, CW), jnp.float32))
    kfn(vf, bcol, ycol, xcol, canvas_ref)
    return canvas_ref[...]


def _sc_fill_scatter_b(vf, bcol, ycol, xcol):
    mesh = plsc.VectorSubcoreMesh(core_axis_name="c", subcore_axis_name="s")
    kfn = pl.kernel(
        _sc_fill_scatter_b_body,
        mesh=mesh,
        out_type=jax.ShapeDtypeStruct((B_ROWS + PAD, CW), jnp.float32),
        compiler_params=pltpu.CompilerParams(use_tc_tiling_on_sc=False),
        scratch_types=[
            pltpu.VMEM((KB, 3, CHUNK), jnp.int32),
            pltpu.VMEM((WAVE, CHUNK, CW), jnp.float32),
            pltpu.VMEM((KB, CHUNK), jnp.int32),
            pltpu.VMEM((FILL_ROWS, CW), jnp.float32),
            pltpu.SemaphoreType.DMA,
            pltpu.SemaphoreType.DMA,
            pltpu.SemaphoreType.DMA,
        ],
    )
    return kfn(vf, bcol, ycol, xcol)


YB = 64


def _tc_transpose_body(x_ref, o_ref):
    for y in range(YB):
        o_ref[:, y, :] = jnp.transpose(x_ref[pl.ds(y * NX, NX), :C], (1, 0))


def _tc_transpose_b(canvas_b):
    return pl.pallas_call(
        _tc_transpose_body,
        grid=(NY // YB,),
        in_specs=[pl.BlockSpec((YB * NX, CW), lambda y: (y, 0))],
        out_specs=pl.BlockSpec((C, YB, NX), lambda y: (3, y, 0)),
        out_shape=jax.ShapeDtypeStruct((B * C, NY, NX), jnp.float32),
    )(canvas_b)


def _tc_transpose_a_body(x_ref, _, o_ref):
    _tc_transpose_body(x_ref, o_ref)


def _tc_transpose_a(canvas_a, out_partial):
    nyb = NY // YB
    return pl.pallas_call(
        _tc_transpose_a_body,
        grid=(3, nyb),
        in_specs=[pl.BlockSpec((YB * NX, CW), lambda b, y: (b * nyb + y, 0)),
                  pl.BlockSpec(memory_space=pl.ANY)],
        out_specs=pl.BlockSpec((C, YB, NX), lambda b, y: (b, y, 0)),
        out_shape=jax.ShapeDtypeStruct((B * C, NY, NX), jnp.float32),
        input_output_aliases={1: 0},
    )(canvas_a, out_partial)


def kernel(voxel_features, coords, batch_size, input_shape):
    del batch_size, input_shape
    bcol, ycol, xcol = coords[:, 0], coords[:, 2], coords[:, 3]
    canvas_b = _sc_fill_scatter_b(voxel_features, bcol, ycol, xcol)
    canvas_a = _sc_scatter_a(voxel_features, bcol, ycol, xcol)
    out = _tc_transpose_b(canvas_b)
    out = _tc_transpose_a(canvas_a, out)
    return out.reshape(B, C, NY, NX)


# speedup vs baseline: 7.1198x; 7.1198x over previous
import jax
import jax.numpy as jnp
from jax import lax
from jax.experimental import pallas as pl
from jax.experimental.pallas import tpu as pltpu
from jax.experimental.pallas import tpu_sc as plsc

B = 4
NY = 512
NX = 512
C = 64
S = B * NY * NX
P = 40000
L = 16
NC = 2
NS = 16
NW = NC * NS
CW = 128
CHUNK = 128
NUM_CHUNKS = (P + NW * CHUNK - 1) // (NW * CHUNK) * NW
WAVE = 5

A_ROWS = 3 * S // 4
B_ROWS = S // 4
PAD = 4096
KA = NUM_CHUNKS // NW
KB = NUM_CHUNKS // NS

FILL_ROWS = 128
TILE_B_ROWS = B_ROWS // NC // NS


def _lin_group(cbuf, k, g):
    bv = cbuf[k, 0, pl.ds(g * L, L)]
    yv = cbuf[k, 1, pl.ds(g * L, L)]
    xv = cbuf[k, 2, pl.ds(g * L, L)]
    return jnp.minimum(bv, B - 1) * (NY * NX) + yv * NX + xv


def _stage_and_scatter(vf_hbm, cols, out_hbm, cbuf, dbuf, ibuf, sem_in, sem_sc,
                       nchunks, chunk_of_k, route):
    for wave in range(nchunks // WAVE):
        in_copies = []
        for kk in range(WAVE):
            k = wave * WAVE + kk
            chunk = chunk_of_k(k)
            start = jnp.minimum(chunk * CHUNK, P - CHUNK)
            for j, col in enumerate(cols):
                in_copies.append(
                    pltpu.async_copy(col.at[pl.ds(start, CHUNK)],
                                     cbuf.at[k, j], sem_in))
            in_copies.append(
                pltpu.async_copy(vf_hbm.at[pl.ds(start, CHUNK)],
                                 dbuf.at[kk, :, pl.ds(0, C)], sem_in))
        for cp in in_copies:
            cp.wait()

        sc_copies = []
        for kk in range(WAVE):
            k = wave * WAVE + kk
            for g in range(CHUNK // L):
                ibuf[k, pl.ds(g * L, L)] = route(_lin_group(cbuf, k, g))
            sc_copies.append(
                pltpu.async_copy(dbuf.at[kk], out_hbm.at[ibuf.at[k]], sem_sc))
        for cp in sc_copies:
            cp.wait()


def _sc_scatter_a_body(vf_hbm, b_hbm, y_hbm, x_hbm, out_hbm,
                       cbuf, dbuf, ibuf, sem_in, sem_sc):
    cid = lax.axis_index("c")
    sid = lax.axis_index("s")
    w = sid * NC + cid

    def route(lin):
        return jnp.where(lin < A_ROWS, lin, A_ROWS + (lin & (PAD - 1)))

    _stage_and_scatter(vf_hbm, (b_hbm, y_hbm, x_hbm), out_hbm,
                       cbuf, dbuf, ibuf, sem_in, sem_sc,
                       KA, lambda k: w + NW * k, route)


def _sc_fill_scatter_b_body(vf_hbm, b_hbm, y_hbm, x_hbm, out_hbm,
                            cbuf, dbuf, ibuf, zbuf, sem_in, sem_sc, sem_z):
    cid = lax.axis_index("c")
    sid = lax.axis_index("s")

    def zrow(j, _):
        for g in range(CW // L):
            zbuf[j, pl.ds(g * L, L)] = jnp.zeros((L,), jnp.float32)
        return _
    lax.fori_loop(0, FILL_ROWS, zrow, None)
    base = (cid * NS + sid) * TILE_B_ROWS
    for grp in range(TILE_B_ROWS // FILL_ROWS // 8):
        cps = [pltpu.async_copy(
                   zbuf,
                   out_hbm.at[pl.ds(base + (grp * 8 + i) * FILL_ROWS,
                                    FILL_ROWS)],
                   sem_z)
               for i in range(8)]
        for cp in cps:
            cp.wait()

    plsc.subcore_barrier()

    half = B_ROWS // NC
    lo = cid * half

    def route(lin):
        lin_b = lin - A_ROWS
        owned = (lin_b >= lo) & (lin_b < lo + half)
        return jnp.where(owned, lin_b, B_ROWS + (lin & (PAD - 1)))

    _stage_and_scatter(vf_hbm, (b_hbm, y_hbm, x_hbm), out_hbm,
                       cbuf, dbuf, ibuf, sem_in, sem_sc,
                       KB, lambda k: sid + NS * k, route)


def _sc_scatter_a(vf, bcol, ycol, xcol):
    mesh = plsc.VectorSubcoreMesh(core_axis_name="c", subcore_axis_name="s")
    kfn = pl.kernel(
        _sc_scatter_a_body,
        mesh=mesh,
        out_type=(),
        compiler_params=pltpu.CompilerParams(use_tc_tiling_on_sc=False),
        scratch_types=[
            pltpu.VMEM((KA, 3, CHUNK), jnp.int32),
            pltpu.VMEM((WAVE, CHUNK, CW), jnp.float32),
            pltpu.VMEM((KA, CHUNK), jnp.int32),
            pltpu.SemaphoreType.DMA,
            pltpu.SemaphoreType.DMA,
        ],
    )
    canvas_ref = jax.new_ref(jnp.zeros((A_ROWS + PAD, CW), jnp.float32))
    kfn(vf, bcol, ycol, xcol, canvas_ref)
    return canvas_ref[...]


def _sc_fill_scatter_b(vf, bcol, ycol, xcol):
    mesh = plsc.VectorSubcoreMesh(core_axis_name="c", subcore_axis_name="s")
    kfn = pl.kernel(
        _sc_fill_scatter_b_body,
        mesh=mesh,
        out_type=jax.ShapeDtypeStruct((B_ROWS + PAD, CW), jnp.float32),
        compiler_params=pltpu.CompilerParams(use_tc_tiling_on_sc=False),
        scratch_types=[
            pltpu.VMEM((KB, 3, CHUNK), jnp.int32),
            pltpu.VMEM((WAVE, CHUNK, CW), jnp.float32),
            pltpu.VMEM((KB, CHUNK), jnp.int32),
            pltpu.VMEM((FILL_ROWS, CW), jnp.float32),
            pltpu.SemaphoreType.DMA,
            pltpu.SemaphoreType.DMA,
            pltpu.SemaphoreType.DMA,
        ],
    )
    return kfn(vf, bcol, ycol, xcol)


YB = 64


def _tc_transpose_body(x_ref, o_ref):
    for y in range(YB):
        o_ref[:, y, :] = jnp.transpose(x_ref[pl.ds(y * NX, NX), :C], (1, 0))


def _tc_transpose_b(canvas_b):
    return pl.pallas_call(
        _tc_transpose_body,
        grid=(NY // YB,),
        in_specs=[pl.BlockSpec((YB * NX, CW), lambda y: (y, 0))],
        out_specs=pl.BlockSpec((C, YB, NX), lambda y: (3, y, 0)),
        out_shape=jax.ShapeDtypeStruct((B * C, NY, NX), jnp.float32),
    )(canvas_b)


def _tc_transpose_a_body(x_ref, _, o_ref):
    _tc_transpose_body(x_ref, o_ref)


def _tc_transpose_a(canvas_a, out_partial):
    nyb = NY // YB
    return pl.pallas_call(
        _tc_transpose_a_body,
        grid=(3, nyb),
        in_specs=[pl.BlockSpec((YB * NX, CW), lambda b, y: (b * nyb + y, 0)),
                  pl.BlockSpec(memory_space=pl.ANY)],
        out_specs=pl.BlockSpec((C, YB, NX), lambda b, y: (b, y, 0)),
        out_shape=jax.ShapeDtypeStruct((B * C, NY, NX), jnp.float32),
        input_output_aliases={1: 0},
    )(canvas_a, out_partial)


def kernel(voxel_features, coords, batch_size, input_shape):
    del batch_size, input_shape
    bcol, ycol, xcol = coords[:, 0], coords[:, 2], coords[:, 3]
    canvas_b = _sc_fill_scatter_b(voxel_features, bcol, ycol, xcol)
    canvas_a = _sc_scatter_a(voxel_features, bcol, ycol, xcol)
    out = _tc_transpose_b(canvas_b)
    out = _tc_transpose_a(canvas_a, out)
    return out.reshape(B, C, NY, NX)


# speedup vs baseline: 7.4061x; 1.0402x over previous
import functools

import jax
import jax.numpy as jnp
from jax import lax
from jax.experimental import pallas as pl
from jax.experimental.pallas import tpu as pltpu
from jax.experimental.pallas import tpu_sc as plsc

B = 4
NY = 512
NX = 512
C = 64
S = B * NY * NX
P = 40000
L = 16
NC = 2
NS = 16
NW = NC * NS
CW = 128
CHUNK = 128
NUM_CHUNKS = (P + NW * CHUNK - 1) // (NW * CHUNK) * NW
KMAX = NUM_CHUNKS // NW
WAVE = 5


def _sc_scatter_body(vf_hbm, b_hbm, y_hbm, x_hbm, out_hbm,
                     cbuf, dbuf, ibuf, sem_in, sem_sc):
    cid = lax.axis_index("c")
    sid = lax.axis_index("s")
    w = sid * NC + cid

    for wave in range(KMAX // WAVE):
        in_copies = []
        for kk in range(WAVE):
            k = wave * WAVE + kk
            chunk = w + NW * k
            start = jnp.minimum(chunk * CHUNK, P - CHUNK)
            for j, col in enumerate((b_hbm, y_hbm, x_hbm)):
                in_copies.append(
                    pltpu.async_copy(col.at[pl.ds(start, CHUNK)],
                                     cbuf.at[k, j], sem_in))
            in_copies.append(
                pltpu.async_copy(vf_hbm.at[pl.ds(start, CHUNK)],
                                 dbuf.at[kk, :, pl.ds(0, C)], sem_in))
        for cp in in_copies:
            cp.wait()

        sc_copies = []
        for kk in range(WAVE):
            k = wave * WAVE + kk
            for g in range(CHUNK // L):
                bv = cbuf[k, 0, pl.ds(g * L, L)]
                yv = cbuf[k, 1, pl.ds(g * L, L)]
                xv = cbuf[k, 2, pl.ds(g * L, L)]
                lin = jnp.minimum(bv, B - 1) * (NY * NX) + yv * NX + xv
                ibuf[k, pl.ds(g * L, L)] = lin
            sc_copies.append(
                pltpu.async_copy(dbuf.at[kk], out_hbm.at[ibuf.at[k]], sem_sc))
        for cp in sc_copies:
            cp.wait()


def _sc_scatter(vf, bcol, ycol, xcol):
    mesh = plsc.VectorSubcoreMesh(core_axis_name="c", subcore_axis_name="s")
    kfn = pl.kernel(
        _sc_scatter_body,
        mesh=mesh,
        out_type=(),
        compiler_params=pltpu.CompilerParams(use_tc_tiling_on_sc=False),
        scratch_types=[
            pltpu.VMEM((KMAX, 3, CHUNK), jnp.int32),
            pltpu.VMEM((WAVE, CHUNK, CW), jnp.float32),
            pltpu.VMEM((KMAX, CHUNK), jnp.int32),
            pltpu.SemaphoreType.DMA,
            pltpu.SemaphoreType.DMA,
        ],
    )
    canvas_ref = jax.new_ref(jnp.zeros((S, CW), jnp.float32))
    kfn(vf, bcol, ycol, xcol, canvas_ref)
    return canvas_ref[...]


YB = 64


def _tc_transpose_body(x_ref, o_ref):
    for y in range(YB):
        o_ref[:, y, :] = jnp.transpose(x_ref[0, y, :, :C], (1, 0))


def _tc_transpose(canvas4):
    return pl.pallas_call(
        _tc_transpose_body,
        grid=(B, NY // YB),
        in_specs=[pl.BlockSpec((1, YB, NX, CW), lambda b, y: (b, y, 0, 0))],
        out_specs=pl.BlockSpec((C, YB, NX), lambda b, y: (b, y, 0)),
        out_shape=jax.ShapeDtypeStruct((B * C, NY, NX), jnp.float32),
    )(canvas4)


def kernel(voxel_features, coords, batch_size, input_shape):
    del batch_size, input_shape
    canvas = _sc_scatter(voxel_features, coords[:, 0], coords[:, 2], coords[:, 3])
    out = _tc_transpose(canvas.reshape(B, NY, NX, CW))
    return out.reshape(B, C, NY, NX)
